# baseline (speedup 1.0000x reference)
import jax
import jax.numpy as jnp
from jax.experimental import pallas as pl
from jax.experimental.pallas import tpu as pltpu

_B, _C, _H, _W = 64, 128, 112, 112
_OH, _OW = 56, 56
_WT = _W // 8


def _quant(v):
    return v.astype(jnp.bfloat16).astype(jnp.float32)


_HB = 56
_OHB = _HB // 2


def _pool_body(x_ref, o_ref):
    v = x_ref[...].reshape(_OHB, 2, _WT, 8, _C)
    r0 = v[:, 0]
    r1 = v[:, 1]
    r0s = pltpu.roll(r0, 7, 2)
    r1s = pltpu.roll(r1, 7, 2)
    y = _quant(r0)
    y = _quant(y + r0s)
    y = _quant(y + r1)
    y = _quant(y + r1s)
    res = _quant(y * 0.25)
    resp = res.reshape(_OHB, _WT // 2, 2, 8, _C)
    si = jax.lax.broadcasted_iota(jnp.int32, (_OHB, _WT // 2, 2, 8, _C), 3)
    g = jnp.take_along_axis(resp, 2 * (si % 4), axis=3)
    si4 = jax.lax.broadcasted_iota(jnp.int32, (_OHB, _WT // 2, 8, _C), 2)
    out = jnp.where(si4 < 4, g[:, :, 0], g[:, :, 1])
    o_ref[...] = out.reshape(1, _OHB, _OW, _C)


def kernel(x):
    xt = jnp.transpose(x, (0, 2, 3, 1))
    out = pl.pallas_call(
        _pool_body,
        grid=(_B * _H // _HB,),
        in_specs=[
            pl.BlockSpec(
                (1, _HB, _W, _C),
                lambda i: (i // (_H // _HB), i % (_H // _HB), 0, 0),
            )
        ],
        out_specs=pl.BlockSpec(
            (1, _OHB, _OW, _C),
            lambda i: (i // (_H // _HB), i % (_H // _HB), 0, 0),
        ),
        out_shape=jax.ShapeDtypeStruct((_B, _OH, _OW, _C), jnp.float32),
        compiler_params=pltpu.CompilerParams(
            dimension_semantics=("parallel",),
        ),
    )(xt)
    return jnp.transpose(out, (0, 3, 1, 2))


# speedup vs baseline: 1.3156x; 1.3156x over previous
import jax
import jax.numpy as jnp
from jax.experimental import pallas as pl
from jax.experimental.pallas import tpu as pltpu

_B, _C, _H, _W = 64, 128, 112, 112
_OH, _OW = 56, 56
_WT = _W // 8


def _quant(v):
    return v.astype(jnp.bfloat16).astype(jnp.float32)


_GB = 2


def _pool_body(x_ref, o_ref):
    v = x_ref[...].reshape(_GB * _OH, 2, _WT, 8, _C)
    r0 = v[:, 0]
    r1 = v[:, 1]
    r0s = pltpu.roll(r0, 7, 2)
    r1s = pltpu.roll(r1, 7, 2)
    y = _quant(r0)
    y = _quant(y + r0s)
    y = _quant(y + r1)
    y = _quant(y + r1s)
    res = _quant(y * 0.25)
    resp = res.reshape(_GB * _OH, _WT // 2, 2, 8, _C)
    si = jax.lax.broadcasted_iota(jnp.int32, (_GB * _OH, _WT // 2, 2, 8, _C), 3)
    g = jnp.take_along_axis(resp, 2 * (si % 4), axis=3)
    si4 = jax.lax.broadcasted_iota(jnp.int32, (_GB * _OH, _WT // 2, 8, _C), 2)
    out = jnp.where(si4 < 4, g[:, :, 0], g[:, :, 1])
    o_ref[...] = out.reshape(_GB, _OH, _OW, _C)


def kernel(x):
    xt = jnp.transpose(x, (0, 2, 3, 1))
    out = pl.pallas_call(
        _pool_body,
        grid=(_B // _GB,),
        in_specs=[pl.BlockSpec((_GB, _H, _W, _C), lambda i: (i, 0, 0, 0))],
        out_specs=pl.BlockSpec((_GB, _OH, _OW, _C), lambda i: (i, 0, 0, 0)),
        out_shape=jax.ShapeDtypeStruct((_B, _OH, _OW, _C), jnp.float32),
        compiler_params=pltpu.CompilerParams(
            dimension_semantics=("parallel",),
        ),
    )(xt)
    return jnp.transpose(out, (0, 3, 1, 2))
